# baseline (device time: 11484 ns/iter reference)
import jax
import jax.numpy as jnp
from jax import lax
from jax.experimental import pallas as pl
from jax.experimental.pallas import tpu as pltpu

C = 4


def kernel(x):
    m, n = x.shape
    rows = m // C

    def body(x_ref, out_ref, send_buf, recv_buf, sum_buf,
             send_sems, recv_sems, out_sems):
        my_x = lax.axis_index("x")
        my_y = lax.axis_index("y")
        my_z = lax.axis_index("z")
        partner = (my_x, 1 - my_y, my_z)

        barrier_sem = pltpu.get_barrier_semaphore()
        pl.semaphore_signal(
            barrier_sem, inc=1,
            device_id=partner, device_id_type=pl.DeviceIdType.MESH,
        )
        pl.semaphore_wait(barrier_sem, 1)

        rdmas = []
        for i in range(C):
            send_buf[i] = x_ref[pl.ds(i * rows, rows), :].astype(jnp.bfloat16)
            rdma = pltpu.make_async_remote_copy(
                src_ref=send_buf.at[i],
                dst_ref=recv_buf.at[i],
                send_sem=send_sems.at[i],
                recv_sem=recv_sems.at[i],
                device_id=partner,
                device_id_type=pl.DeviceIdType.MESH,
            )
            rdma.start()
            rdmas.append(rdma)

        stores = []
        for i in range(C):
            rdmas[i].wait_recv()
            sum_buf[i] = (
                x_ref[pl.ds(i * rows, rows), :]
                + recv_buf[i].astype(jnp.float32)
            )
            store = pltpu.make_async_copy(
                sum_buf.at[i],
                out_ref.at[pl.ds(i * rows, rows), :],
                out_sems.at[i],
            )
            store.start()
            stores.append(store)

        for i in range(C):
            stores[i].wait()
            rdmas[i].wait_send()

    return pl.pallas_call(
        body,
        out_shape=jax.ShapeDtypeStruct((m, n), jnp.float32),
        in_specs=[pl.BlockSpec(memory_space=pltpu.VMEM)],
        out_specs=pl.BlockSpec(memory_space=pl.ANY),
        scratch_shapes=[
            pltpu.VMEM((C, rows, n), jnp.bfloat16),
            pltpu.VMEM((C, rows, n), jnp.bfloat16),
            pltpu.VMEM((C, rows, n), jnp.float32),
            pltpu.SemaphoreType.DMA((C,)),
            pltpu.SemaphoreType.DMA((C,)),
            pltpu.SemaphoreType.DMA((C,)),
        ],
        compiler_params=pltpu.CompilerParams(collective_id=0),
    )(x)
